# baseline (speedup 1.0000x reference)
import jax
import jax.numpy as jnp
from jax import lax
from jax.experimental import pallas as pl
from jax.experimental.pallas import tpu as pltpu
from jax.experimental.pallas import tpu_sc as plsc

N = 3200000
G = 100000
GPAD = 102400
BLK = 640000

NUM_SC = 2
TILES = 16
NUM_W = NUM_SC * TILES
PER_W = N // NUM_W
CH = 25000
NCH = PER_W // CH
SLICE = GPAD // TILES


def _vals_body(x_ref, y_ref, z_ref, vals_ref):
    x = x_ref[...]
    y = y_ref[...]
    z = z_ref[...]
    vals_ref[...] = (x * x + y * y + z * z) * 0.25


_vals_call = pl.pallas_call(
    _vals_body,
    grid=(N // BLK,),
    in_specs=[
        pl.BlockSpec((BLK,), lambda i: (i,)),
        pl.BlockSpec((BLK,), lambda i: (i,)),
        pl.BlockSpec((BLK,), lambda i: (i,)),
    ],
    out_specs=pl.BlockSpec((BLK,), lambda i: (i,)),
    out_shape=jax.ShapeDtypeStruct((N,), jnp.float32),
)


def _charges_body(x_ref, y_ref, z_ref, charges_ref):
    charges_ref[...] = x_ref[...] + y_ref[...] + z_ref[...]


_charges_call = pl.pallas_call(
    _charges_body,
    grid=(N // BLK,),
    in_specs=[
        pl.BlockSpec((BLK,), lambda i: (i,)),
        pl.BlockSpec((BLK,), lambda i: (i,)),
        pl.BlockSpec((BLK,), lambda i: (i,)),
    ],
    out_specs=pl.BlockSpec((BLK,), lambda i: (i,)),
    out_shape=jax.ShapeDtypeStruct((N,), jnp.float32),
)


def _scatter_body(vals_hbm, batch_hbm, out_hbm, idx0_v, idx1_v, val0_v,
                  val1_v, buf_v, acc, sem_i, sem_v):
    cid = lax.axis_index("c")
    sid = lax.axis_index("s")
    wid = cid * TILES + sid
    idx_bufs = (idx0_v, idx1_v)
    val_bufs = (val0_v, val1_v)

    def _start_load(k):
        base = wid * PER_W + k * CH
        b = k % 2
        return (
            pltpu.async_copy(batch_hbm.at[pl.ds(base, CH)], idx_bufs[b],
                             sem_i.at[b]),
            pltpu.async_copy(vals_hbm.at[pl.ds(base, CH)], val_bufs[b],
                             sem_v.at[b]),
        )

    handles = {0: _start_load(0)}

    def _zero(i, carry):
        buf_v[pl.ds(i * 16, 16)] = jnp.zeros((16,), jnp.float32)
        return carry

    lax.fori_loop(0, SLICE // 16, _zero, 0)
    pltpu.sync_copy(buf_v, acc.at[pl.ds(sid * SLICE, SLICE)])
    plsc.subcore_barrier()

    for k in range(NCH):
        if k + 1 < NCH:
            handles[k + 1] = _start_load(k + 1)
        hi, hv = handles.pop(k)
        hi.wait()
        hv.wait()
        b = k % 2
        pltpu.sync_copy(val_bufs[b], acc.at[idx_bufs[b]], add=True)
    plsc.subcore_barrier()

    pltpu.sync_copy(acc.at[pl.ds(sid * SLICE, SLICE)], buf_v)
    pltpu.sync_copy(buf_v, out_hbm.at[cid, pl.ds(sid * SLICE, SLICE)])


_scatter_call = pl.kernel(
    _scatter_body,
    out_type=jax.ShapeDtypeStruct((NUM_SC, GPAD), jnp.float32),
    mesh=plsc.VectorSubcoreMesh(core_axis_name="c", subcore_axis_name="s"),
    scratch_types=[
        pltpu.VMEM((CH,), jnp.int32),
        pltpu.VMEM((CH,), jnp.int32),
        pltpu.VMEM((CH,), jnp.float32),
        pltpu.VMEM((CH,), jnp.float32),
        pltpu.VMEM((SLICE,), jnp.float32),
        pltpu.VMEM_SHARED((GPAD,), jnp.float32),
        pltpu.SemaphoreType.DMA((2,)),
        pltpu.SemaphoreType.DMA((2,)),
    ],
)


def _combine_body(p_ref, out_ref):
    out_ref[...] = p_ref[0] + p_ref[1]


_combine_call = pl.pallas_call(
    _combine_body,
    in_specs=[pl.BlockSpec((NUM_SC, GPAD // 128, 128), lambda: (0, 0, 0))],
    out_specs=pl.BlockSpec((GPAD // 128, 128), lambda: (0, 0)),
    out_shape=jax.ShapeDtypeStruct((GPAD // 128, 128), jnp.float32),
)


def kernel(positions, atomic_numbers, batch):
    del atomic_numbers
    x = positions[:, 0]
    y = positions[:, 1]
    z = positions[:, 2]
    vals_flat = _vals_call(x, y, z)
    partials = _scatter_call(vals_flat, batch.astype(jnp.int32))
    charges_flat = _charges_call(x, y, z)
    combined = _combine_call(partials.reshape(NUM_SC, GPAD // 128, 128))
    energies = combined.reshape(GPAD)[:G].reshape(G, 1)
    node_charges = charges_flat.reshape(N, 1)
    return (energies, node_charges)


# speedup vs baseline: 1.1099x; 1.1099x over previous
import jax
import jax.numpy as jnp
from jax import lax
from jax.experimental import pallas as pl
from jax.experimental.pallas import tpu as pltpu
from jax.experimental.pallas import tpu_sc as plsc

N = 3200000
G = 100000
GPAD = 102400
BLK = 640000

NUM_SC = 2
TILES = 16
NUM_W = NUM_SC * TILES
PER_W = N // NUM_W
CH = 10000
NCH = PER_W // CH
SLICE = GPAD // TILES


def _charges_body(x_ref, y_ref, z_ref, charges_ref):
    charges_ref[...] = x_ref[...] + y_ref[...] + z_ref[...]


_charges_call = pl.pallas_call(
    _charges_body,
    grid=(N // BLK,),
    in_specs=[
        pl.BlockSpec((BLK,), lambda i: (i,)),
        pl.BlockSpec((BLK,), lambda i: (i,)),
        pl.BlockSpec((BLK,), lambda i: (i,)),
    ],
    out_specs=pl.BlockSpec((BLK,), lambda i: (i,)),
    out_shape=jax.ShapeDtypeStruct((N,), jnp.float32),
)


def _scatter_body(x_hbm, y_hbm, z_hbm, batch_hbm, out_hbm,
                  ix0, ix1, ix2, xb0, xb1, yb0, yb1, zb0, zb1, vb0, vb1,
                  buf_v, acc, sem_i, sem_x, sem_y, sem_z, sem_s):
    cid = lax.axis_index("c")
    sid = lax.axis_index("s")
    wid = cid * TILES + sid
    ibufs = (ix0, ix1, ix2)
    xbufs = (xb0, xb1)
    ybufs = (yb0, yb1)
    zbufs = (zb0, zb1)
    vbufs = (vb0, vb1)

    def _start_load(k):
        base = wid * PER_W + k * CH
        b2 = k % 2
        b3 = k % 3
        return (
            pltpu.async_copy(batch_hbm.at[pl.ds(base, CH)], ibufs[b3],
                             sem_i.at[b3]),
            pltpu.async_copy(x_hbm.at[pl.ds(base, CH)], xbufs[b2],
                             sem_x.at[b2]),
            pltpu.async_copy(y_hbm.at[pl.ds(base, CH)], ybufs[b2],
                             sem_y.at[b2]),
            pltpu.async_copy(z_hbm.at[pl.ds(base, CH)], zbufs[b2],
                             sem_z.at[b2]),
        )

    handles = {0: _start_load(0)}

    def _zero(i, carry):
        buf_v[pl.ds(i * 16, 16)] = jnp.zeros((16,), jnp.float32)
        return carry

    lax.fori_loop(0, SLICE // 16, _zero, 0)
    pltpu.sync_copy(buf_v, acc.at[pl.ds(sid * SLICE, SLICE)])
    plsc.subcore_barrier()

    scat = {}
    for k in range(NCH):
        if k >= 2:
            scat.pop(k - 2).wait()
        if k + 1 < NCH:
            handles[k + 1] = _start_load(k + 1)
        hi, hx, hy, hz = handles.pop(k)
        hi.wait()
        hx.wait()
        hy.wait()
        hz.wait()
        b2 = k % 2
        b3 = k % 3
        xb, yb, zb, vb = xbufs[b2], ybufs[b2], zbufs[b2], vbufs[b2]

        def _compute(i, carry):
            xv = xb[pl.ds(i * 16, 16)]
            yv = yb[pl.ds(i * 16, 16)]
            zv = zb[pl.ds(i * 16, 16)]
            vb[pl.ds(i * 16, 16)] = (xv * xv + yv * yv + zv * zv) * 0.25
            return carry

        lax.fori_loop(0, CH // 16, _compute, 0)
        scat[k] = pltpu.async_copy(vb, acc.at[ibufs[b3]], add=True,
                                   sem=sem_s.at[b2])
    for k in sorted(scat):
        scat.pop(k).wait()
    plsc.subcore_barrier()

    pltpu.sync_copy(acc.at[pl.ds(sid * SLICE, SLICE)], buf_v)
    pltpu.sync_copy(buf_v, out_hbm.at[cid, pl.ds(sid * SLICE, SLICE)])


_scatter_call = pl.kernel(
    _scatter_body,
    out_type=jax.ShapeDtypeStruct((NUM_SC, GPAD), jnp.float32),
    mesh=plsc.VectorSubcoreMesh(core_axis_name="c", subcore_axis_name="s"),
    scratch_types=[
        pltpu.VMEM((CH,), jnp.int32),
        pltpu.VMEM((CH,), jnp.int32),
        pltpu.VMEM((CH,), jnp.int32),
        pltpu.VMEM((CH,), jnp.float32),
        pltpu.VMEM((CH,), jnp.float32),
        pltpu.VMEM((CH,), jnp.float32),
        pltpu.VMEM((CH,), jnp.float32),
        pltpu.VMEM((CH,), jnp.float32),
        pltpu.VMEM((CH,), jnp.float32),
        pltpu.VMEM((CH,), jnp.float32),
        pltpu.VMEM((CH,), jnp.float32),
        pltpu.VMEM((SLICE,), jnp.float32),
        pltpu.VMEM_SHARED((GPAD,), jnp.float32),
        pltpu.SemaphoreType.DMA((3,)),
        pltpu.SemaphoreType.DMA((2,)),
        pltpu.SemaphoreType.DMA((2,)),
        pltpu.SemaphoreType.DMA((2,)),
        pltpu.SemaphoreType.DMA((2,)),
    ],
)


def _combine_body(p_ref, out_ref):
    out_ref[...] = p_ref[0] + p_ref[1]


_combine_call = pl.pallas_call(
    _combine_body,
    in_specs=[pl.BlockSpec((NUM_SC, GPAD // 128, 128), lambda: (0, 0, 0))],
    out_specs=pl.BlockSpec((GPAD // 128, 128), lambda: (0, 0)),
    out_shape=jax.ShapeDtypeStruct((GPAD // 128, 128), jnp.float32),
)


def kernel(positions, atomic_numbers, batch):
    del atomic_numbers
    x = positions[:, 0]
    y = positions[:, 1]
    z = positions[:, 2]
    partials = _scatter_call(x, y, z, batch.astype(jnp.int32))
    charges_flat = _charges_call(x, y, z)
    combined = _combine_call(partials.reshape(NUM_SC, GPAD // 128, 128))
    energies = combined.reshape(GPAD)[:G].reshape(G, 1)
    node_charges = charges_flat.reshape(N, 1)
    return (energies, node_charges)


# speedup vs baseline: 1.1248x; 1.0134x over previous
import jax
import jax.numpy as jnp
from jax import lax
from jax.experimental import pallas as pl
from jax.experimental.pallas import tpu as pltpu
from jax.experimental.pallas import tpu_sc as plsc

N = 3200000
G = 100000
GPAD = 102400
BLK = 640000

NUM_SC = 2
TILES = 16
NUM_W = NUM_SC * TILES
PER_W = N // NUM_W
CH = 10000
NCH = PER_W // CH
SLICE = GPAD // TILES


def _charges_body(x_ref, y_ref, z_ref, charges_ref):
    charges_ref[...] = x_ref[...] + y_ref[...] + z_ref[...]


_charges_call = pl.pallas_call(
    _charges_body,
    grid=(N // BLK,),
    in_specs=[
        pl.BlockSpec((BLK,), lambda i: (i,)),
        pl.BlockSpec((BLK,), lambda i: (i,)),
        pl.BlockSpec((BLK,), lambda i: (i,)),
    ],
    out_specs=pl.BlockSpec((BLK,), lambda i: (i,)),
    out_shape=jax.ShapeDtypeStruct((N,), jnp.float32),
)


def _scatter_body(x_hbm, y_hbm, z_hbm, batch_hbm, out_hbm,
                  ix0, ix1, ix2, xb0, xb1, yb0, yb1, zb0, zb1, vb0, vb1,
                  buf_v, acc, sem_i, sem_x, sem_y, sem_z, sem_s):
    cid = lax.axis_index("c")
    sid = lax.axis_index("s")
    wid = cid * TILES + sid
    ibufs = (ix0, ix1, ix2)
    xbufs = (xb0, xb1)
    ybufs = (yb0, yb1)
    zbufs = (zb0, zb1)
    vbufs = (vb0, vb1)

    def _start_load(k):
        base = wid * PER_W + k * CH
        b2 = k % 2
        b3 = k % 3
        return (
            pltpu.async_copy(batch_hbm.at[pl.ds(base, CH)], ibufs[b3],
                             sem_i.at[b3]),
            pltpu.async_copy(x_hbm.at[pl.ds(base, CH)], xbufs[b2],
                             sem_x.at[b2]),
            pltpu.async_copy(y_hbm.at[pl.ds(base, CH)], ybufs[b2],
                             sem_y.at[b2]),
            pltpu.async_copy(z_hbm.at[pl.ds(base, CH)], zbufs[b2],
                             sem_z.at[b2]),
        )

    handles = {0: _start_load(0), 1: _start_load(1)}

    def _zero(i, carry):
        buf_v[pl.ds(i * 16, 16)] = jnp.zeros((16,), jnp.float32)
        return carry

    lax.fori_loop(0, SLICE // 16, _zero, 0)
    pltpu.sync_copy(buf_v, acc.at[pl.ds(sid * SLICE, SLICE)])
    plsc.subcore_barrier()

    scat = {}
    for k in range(NCH):
        if k >= 2:
            scat.pop(k - 2).wait()
        if 2 <= k + 1 < NCH:
            handles[k + 1] = _start_load(k + 1)
        hi, hx, hy, hz = handles.pop(k)
        hi.wait()
        hx.wait()
        hy.wait()
        hz.wait()
        b2 = k % 2
        b3 = k % 3
        xb, yb, zb, vb = xbufs[b2], ybufs[b2], zbufs[b2], vbufs[b2]

        def _compute(i, carry):
            for u in range(4):
                off = i * 64 + u * 16
                xv = xb[pl.ds(off, 16)]
                yv = yb[pl.ds(off, 16)]
                zv = zb[pl.ds(off, 16)]
                vb[pl.ds(off, 16)] = (xv * xv + yv * yv + zv * zv) * 0.25
            return carry

        lax.fori_loop(0, CH // 64, _compute, 0)
        scat[k] = pltpu.async_copy(vb, acc.at[ibufs[b3]], add=True,
                                   sem=sem_s.at[b2])
    for k in sorted(scat):
        scat.pop(k).wait()
    plsc.subcore_barrier()

    pltpu.sync_copy(acc.at[pl.ds(sid * SLICE, SLICE)], buf_v)
    pltpu.sync_copy(buf_v, out_hbm.at[cid, pl.ds(sid * SLICE, SLICE)])


_scatter_call = pl.kernel(
    _scatter_body,
    out_type=jax.ShapeDtypeStruct((NUM_SC, GPAD), jnp.float32),
    mesh=plsc.VectorSubcoreMesh(core_axis_name="c", subcore_axis_name="s"),
    scratch_types=[
        pltpu.VMEM((CH,), jnp.int32),
        pltpu.VMEM((CH,), jnp.int32),
        pltpu.VMEM((CH,), jnp.int32),
        pltpu.VMEM((CH,), jnp.float32),
        pltpu.VMEM((CH,), jnp.float32),
        pltpu.VMEM((CH,), jnp.float32),
        pltpu.VMEM((CH,), jnp.float32),
        pltpu.VMEM((CH,), jnp.float32),
        pltpu.VMEM((CH,), jnp.float32),
        pltpu.VMEM((CH,), jnp.float32),
        pltpu.VMEM((CH,), jnp.float32),
        pltpu.VMEM((SLICE,), jnp.float32),
        pltpu.VMEM_SHARED((GPAD,), jnp.float32),
        pltpu.SemaphoreType.DMA((3,)),
        pltpu.SemaphoreType.DMA((2,)),
        pltpu.SemaphoreType.DMA((2,)),
        pltpu.SemaphoreType.DMA((2,)),
        pltpu.SemaphoreType.DMA((2,)),
    ],
)


def _combine_body(p_ref, out_ref):
    out_ref[...] = p_ref[0] + p_ref[1]


_combine_call = pl.pallas_call(
    _combine_body,
    in_specs=[pl.BlockSpec((NUM_SC, GPAD // 128, 128), lambda: (0, 0, 0))],
    out_specs=pl.BlockSpec((GPAD // 128, 128), lambda: (0, 0)),
    out_shape=jax.ShapeDtypeStruct((GPAD // 128, 128), jnp.float32),
)


def kernel(positions, atomic_numbers, batch):
    del atomic_numbers
    x = positions[:, 0]
    y = positions[:, 1]
    z = positions[:, 2]
    partials = _scatter_call(x, y, z, batch.astype(jnp.int32))
    charges_flat = _charges_call(x, y, z)
    combined = _combine_call(partials.reshape(NUM_SC, GPAD // 128, 128))
    energies = combined.reshape(GPAD)[:G].reshape(G, 1)
    node_charges = charges_flat.reshape(N, 1)
    return (energies, node_charges)
